# Initial kernel scaffold; baseline (speedup 1.0000x reference)
import jax
import jax.numpy as jnp
from jax.experimental import pallas as pl


def kernel(nodes, edges, receivers, senders, node_graph_idx, edge_graph_idx, emb_0, emb_1, emb_2, emb_3, emb_4, emb_5, emb_6, emb_7, emb_8):
    raise NotImplementedError("write your pallas kernel here")


# speedup vs baseline: 1.3846x; 1.3846x over previous
import functools

import jax
import jax.numpy as jnp
from jax import lax
from jax.experimental import pallas as pl
from jax.experimental.pallas import tpu as pltpu
from jax.experimental.pallas import tpu_sc as plsc

D_EMB = 128
N_NODES = 100000
BATCH = 1024

_OFF0 = 0
_OFF1 = 119
_OFF2 = 263
_OFF3 = 623
_TBL_ROWS = 640

_CHUNK_NODES = 160
_N_CHUNKS = N_NODES // _CHUNK_NODES
_GROUPS = _CHUNK_NODES // 16


def _sc_lookup(table, nodes):
    n_cores, n_subcores = 2, 16
    n_workers = n_cores * n_subcores
    iters = (_N_CHUNKS + n_workers - 1) // n_workers

    mesh = plsc.VectorSubcoreMesh(core_axis_name="c", subcore_axis_name="s",
                                  num_cores=n_cores)

    @functools.partial(
        pl.kernel,
        mesh=mesh,
        compiler_params=pltpu.CompilerParams(needs_layout_passes=False),
        out_type=jax.ShapeDtypeStruct((N_NODES, D_EMB), jnp.float32),
        scratch_types=[
            pltpu.VMEM((_TBL_ROWS, D_EMB), jnp.float32),
            pltpu.VMEM((_CHUNK_NODES * 9,), jnp.int32),
            pltpu.VMEM((_CHUNK_NODES, D_EMB), jnp.float32),
        ],
    )
    def body(table_hbm, nodes_hbm, out_hbm, tbl_v, idx_v, out_v):
        wid = lax.axis_index("c") * n_subcores + lax.axis_index("s")
        pltpu.sync_copy(table_hbm, tbl_v)

        iota = jnp.arange(16, dtype=jnp.int32)

        def chunk_body(k, carry):
            c = wid + n_workers * k

            @pl.when(c < _N_CHUNKS)
            def _():
                base = c * _CHUNK_NODES
                pltpu.sync_copy(
                    nodes_hbm.at[pl.ds(base * 9, _CHUNK_NODES * 9)], idx_v)
                for g in range(_GROUPS):
                    rows = iota + (g * 16)
                    flat9 = rows * 9

                    def col(j):
                        return plsc.load_gather(idx_v, [flat9 + j])

                    n0, n1, n2 = col(0), col(1), col(2)
                    n3, n4, n5 = col(3), col(4), col(5)
                    n6, n7, n8 = col(6), col(7), col(8)
                    b0 = n0
                    b1 = n2 * 12 + n3 + _OFF1
                    b2 = (n4 * 6 + n5) * 6 + n6 + _OFF2
                    b3 = (n1 * 2 + n7) * 2 + n8 + _OFF3

                    def d_body(d, carry2):
                        dsp = jnp.full((16,), d, jnp.int32)
                        acc = plsc.load_gather(tbl_v, [b0, dsp])
                        acc = acc + plsc.load_gather(tbl_v, [b1, dsp])
                        acc = acc + plsc.load_gather(tbl_v, [b2, dsp])
                        acc = acc + plsc.load_gather(tbl_v, [b3, dsp])
                        plsc.store_scatter(out_v, [rows, dsp], acc)
                        return carry2

                    lax.fori_loop(0, D_EMB, d_body, 0)
                pltpu.sync_copy(out_v,
                                out_hbm.at[pl.ds(base, _CHUNK_NODES), :])

            return carry

        lax.fori_loop(0, iters, chunk_body, 0)

    return body(table, nodes)


def kernel(nodes, edges, receivers, senders, node_graph_idx, edge_graph_idx,
           emb_0, emb_1, emb_2, emb_3, emb_4, emb_5, emb_6, emb_7, emb_8):
    nodes = nodes.astype(jnp.int32)
    t1 = (emb_2[:, None, :] + emb_3[None, :, :]).reshape(144, D_EMB)
    t2 = (emb_4[:, None, None, :] + emb_5[None, :, None, :]
          + emb_6[None, None, :, :]).reshape(360, D_EMB)
    t3 = (emb_1[:, None, None, :] + emb_7[None, :, None, :]
          + emb_8[None, None, :, :]).reshape(16, D_EMB)
    table = jnp.concatenate(
        [emb_0, t1, t2, t3, jnp.zeros((1, D_EMB), jnp.float32)], axis=0)
    x = _sc_lookup(table, nodes.reshape(-1))
    globals_zero = jnp.zeros((BATCH, 1), dtype=jnp.float32)
    return (x, edges, receivers, senders, globals_zero,
            node_graph_idx, edge_graph_idx)
